# Initial kernel scaffold; baseline (speedup 1.0000x reference)
import jax
import jax.numpy as jnp
from jax.experimental import pallas as pl


def kernel(edges, walks, walks_edge_types, nodes, batch_size, n_positive, repeat_edge_types, u_time_delta, v_time_delta, u_pos_reps_mask, v_pos_reps_mask, u_pos_reps_loss_mask, v_pos_reps_loss_mask, node_emb_w, short_emb_w, edge_embedding, alpha, node_types_arr):
    raise NotImplementedError("write your pallas kernel here")


# speedup vs baseline: 3.1091x; 3.1091x over previous
import functools

import jax
import jax.numpy as jnp
from jax import lax
from jax.experimental import pallas as pl
from jax.experimental.pallas import tpu as pltpu
from jax.experimental.pallas import tpu_sc as plsc

NC = 2
NS = 16
NW = NC * NS
L = 16

N_NEG = 5


def _build_sc_kernel(B, P, NNODES, D, NEG):
    e_pw = B // NW
    pos_pw = (B * P) // NW
    neg_pw = (B * NEG) // NW
    C = 80
    n_pos_chunks = pos_pw // C
    n_neg_chunks = neg_pw // C
    assert pos_pw % C == 0 and neg_pw % C == 0 and B % NW == 0

    mesh = plsc.VectorSubcoreMesh(core_axis_name="c", subcore_axis_name="s")
    f32 = jnp.float32
    i32 = jnp.int32

    out_type = (
        jax.ShapeDtypeStruct((B, D), f32),
        jax.ShapeDtypeStruct((B, D), f32),
        jax.ShapeDtypeStruct((B, D), f32),
        jax.ShapeDtypeStruct((B, D), f32),
        jax.ShapeDtypeStruct((B * P, D), f32),
        jax.ShapeDtypeStruct((B * P, D), f32),
        jax.ShapeDtypeStruct((B * NEG, D), f32),
        jax.ShapeDtypeStruct((B * NEG, D), f32),
    )
    scratch_types = [
        pltpu.VMEM((2048,), i32),
        pltpu.VMEM((e_pw,), i32),
        pltpu.VMEM((e_pw,), i32),
        pltpu.VMEM((e_pw,), i32),
        pltpu.VMEM((e_pw,), f32),
        pltpu.VMEM((e_pw, D), f32),
        pltpu.VMEM((e_pw, D), f32),
        pltpu.VMEM((e_pw, D), f32),
        pltpu.VMEM((e_pw, D), f32),
        pltpu.VMEM((e_pw, D), f32),
        pltpu.VMEM((C,), i32),
        pltpu.VMEM((C,), i32),
        pltpu.VMEM((C,), i32),
        pltpu.VMEM((C, D), f32),
        pltpu.SemaphoreType.DMA,
    ]

    def body(node_emb, short_emb, edge_flat, u_idx, v_idx, et,
             u_dec, v_dec, posu_n, posu_e, posv_n, posv_e,
             u_rand, v_rand, ret, nodes,
             u_reps_o, v_reps_o, u_edge_o, v_edge_o,
             u_pos_o, v_pos_o, u_neg_o, v_neg_o,
             nodes_v, aidx_v, et_v, eidx_v, dec_v,
             nrows, srows, erows, rep_b, repe_b,
             ca_v, cb_v, cf_v, cdata_v, sem):
        wid = lax.axis_index("s") * NC + lax.axis_index("c")

        pltpu.sync_copy(nodes, nodes_v)

        ebase = pl.multiple_of(wid * e_pw, 8)
        pltpu.sync_copy(et.at[pl.ds(ebase, e_pw)], et_v)

        def reps_for(idx_hbm, dec_hbm, reps_hbm, edge_hbm):
            pltpu.sync_copy(idx_hbm.at[pl.ds(ebase, e_pw)], aidx_v)
            pltpu.sync_copy(dec_hbm.at[pl.ds(ebase, e_pw)], dec_v)
            for i in range(e_pw // L):
                sl = pl.ds(i * L, L)
                eidx_v[sl] = aidx_v[sl] * 4 + et_v[sl]
            c1 = pltpu.async_copy(node_emb.at[aidx_v], nrows, sem)
            c2 = pltpu.async_copy(short_emb.at[aidx_v], srows, sem)
            c3 = pltpu.async_copy(edge_flat.at[eidx_v], erows, sem)
            c1.wait(); c2.wait(); c3.wait()

            def row_body(r, carry):
                db = plsc.load_gather(dec_v, [jnp.full((L,), r, i32)])
                for c in range(D // L):
                    sl = pl.ds(c * L, L)
                    ur = nrows[r, sl] + srows[r, sl] * db
                    rep_b[r, sl] = ur
                    repe_b[r, sl] = (ur + erows[r, sl]) * 0.5
                return carry
            lax.fori_loop(0, e_pw, row_body, 0)
            pltpu.sync_copy(rep_b, reps_hbm.at[pl.ds(ebase, e_pw)])
            pltpu.sync_copy(repe_b, edge_hbm.at[pl.ds(ebase, e_pw)])

        reps_for(u_idx, u_dec, u_reps_o, u_edge_o)
        reps_for(v_idx, v_dec, v_reps_o, v_edge_o)

        def gather_seg(n_hbm, e_hbm, out_hbm, rows_pw, nchunks, via_nodes):
            gbase0 = pl.multiple_of(wid * rows_pw, 8)

            def chunk(j, carry):
                gbase = pl.multiple_of(gbase0 + j * C, 8)
                pltpu.sync_copy(n_hbm.at[pl.ds(gbase, C)], ca_v)
                pltpu.sync_copy(e_hbm.at[pl.ds(gbase, C)], cb_v)
                for i in range(C // L):
                    sl = pl.ds(i * L, L)
                    n16 = ca_v[sl]
                    if via_nodes:
                        n16 = plsc.load_gather(nodes_v, [n16])
                    cf_v[sl] = n16 * 4 + cb_v[sl]
                pltpu.async_copy(edge_flat.at[cf_v], cdata_v, sem).wait()
                pltpu.sync_copy(cdata_v, out_hbm.at[pl.ds(gbase, C)])
                return carry
            lax.fori_loop(0, nchunks, chunk, 0)

        gather_seg(posu_n, posu_e, u_pos_o, pos_pw, n_pos_chunks, False)
        gather_seg(posv_n, posv_e, v_pos_o, pos_pw, n_pos_chunks, False)
        gather_seg(u_rand, ret, u_neg_o, neg_pw, n_neg_chunks, True)
        gather_seg(v_rand, ret, v_neg_o, neg_pw, n_neg_chunks, True)

    return pl.kernel(body, out_type=out_type, mesh=mesh,
                     scratch_types=scratch_types,
                     compiler_params=pltpu.CompilerParams(
                         needs_layout_passes=False))


def kernel(edges, walks, walks_edge_types, nodes, batch_size, n_positive,
           repeat_edge_types, u_time_delta, v_time_delta, u_pos_reps_mask,
           v_pos_reps_mask, u_pos_reps_loss_mask, v_pos_reps_loss_mask,
           node_emb_w, short_emb_w, edge_embedding, alpha, node_types_arr):
    B = walks.shape[0]
    P = walks.shape[2] * walks.shape[3]
    NNODES, NET, D = edge_embedding.shape
    NEG = N_NEG * P

    u_idx = edges[:, 0]
    v_idx = edges[:, 1]
    et = edges[:, 2]

    u_sig = jax.nn.sigmoid(alpha[node_types_arr[u_idx]])
    v_sig = jax.nn.sigmoid(alpha[node_types_arr[v_idx]])
    u_dec = 1.0 / jnp.log(2.7183 + u_sig * u_time_delta)
    v_dec = 1.0 / jnp.log(2.7183 + v_sig * v_time_delta)

    posu_n = walks[:, 0].reshape(-1)
    posv_n = walks[:, 1].reshape(-1)
    posu_e = walks_edge_types[:, 0].reshape(-1)
    posv_e = walks_edge_types[:, 1].reshape(-1)

    kneg = jax.random.key(123)
    k1, k2 = jax.random.split(kneg)
    u_rand = jax.random.randint(k1, (B * NEG,), 0, nodes.shape[0])
    v_rand = jax.random.randint(k2, (B * NEG,), 0, nodes.shape[0])
    ret = repeat_edge_types.reshape(-1)

    edge_flat = edge_embedding.reshape(NNODES * NET, D)

    sck = _build_sc_kernel(B, P, NNODES, D, NEG)
    (u_reps, v_reps, u_edge, v_edge, u_pos, v_pos, u_neg, v_neg) = sck(
        node_emb_w, short_emb_w, edge_flat,
        u_idx.astype(jnp.int32), v_idx.astype(jnp.int32),
        et.astype(jnp.int32), u_dec, v_dec,
        posu_n.astype(jnp.int32), posu_e.astype(jnp.int32),
        posv_n.astype(jnp.int32), posv_e.astype(jnp.int32),
        u_rand.astype(jnp.int32), v_rand.astype(jnp.int32),
        ret.astype(jnp.int32), nodes.astype(jnp.int32))

    return (u_reps, v_reps,
            u_pos.reshape(B, P, D), v_pos.reshape(B, P, D),
            u_neg.reshape(B, NEG, D), v_neg.reshape(B, NEG, D),
            n_positive, u_pos_reps_mask, v_pos_reps_mask,
            u_reps, v_reps, u_pos_reps_loss_mask, v_pos_reps_loss_mask,
            u_edge, v_edge)


# speedup vs baseline: 3.8794x; 1.2478x over previous
import functools

import jax
import jax.numpy as jnp
from jax import lax
from jax.experimental import pallas as pl
from jax.experimental.pallas import tpu as pltpu
from jax.experimental.pallas import tpu_sc as plsc

NC = 2
NS = 16
NW = NC * NS
L = 16

N_NEG = 5
NBUF = 4


def _build_sc_kernel(B, P, NNODES, D, NEG):
    e_pw = B // NW
    pos_pw = (B * P) // NW
    neg_pw = (B * NEG) // NW
    C = 80
    n_pos_chunks = pos_pw // C
    n_neg_chunks = neg_pw // C
    assert pos_pw % C == 0 and neg_pw % C == 0 and B % NW == 0
    assert n_pos_chunks % NBUF == 0 and n_neg_chunks % NBUF == 0

    mesh = plsc.VectorSubcoreMesh(core_axis_name="c", subcore_axis_name="s")
    f32 = jnp.float32
    i32 = jnp.int32

    out_type = (
        jax.ShapeDtypeStruct((B, D), f32),
        jax.ShapeDtypeStruct((B, D), f32),
        jax.ShapeDtypeStruct((B, D), f32),
        jax.ShapeDtypeStruct((B, D), f32),
        jax.ShapeDtypeStruct((B * P, D), f32),
        jax.ShapeDtypeStruct((B * P, D), f32),
        jax.ShapeDtypeStruct((B * NEG, D), f32),
        jax.ShapeDtypeStruct((B * NEG, D), f32),
    )
    scratch_types = [
        pltpu.VMEM((2048,), i32),
        pltpu.VMEM((e_pw,), i32),
        pltpu.VMEM((e_pw,), i32),
        pltpu.VMEM((e_pw,), i32),
        pltpu.VMEM((e_pw,), i32),
        pltpu.VMEM((e_pw,), i32),
        pltpu.VMEM((e_pw,), f32),
        pltpu.VMEM((e_pw,), f32),
        pltpu.VMEM((e_pw, D), f32),
        pltpu.VMEM((e_pw, D), f32),
        pltpu.VMEM((e_pw, D), f32),
        pltpu.VMEM((e_pw, D), f32),
        pltpu.VMEM((e_pw, D), f32),
        pltpu.VMEM((e_pw, D), f32),
        pltpu.VMEM((e_pw, D), f32),
        pltpu.VMEM((e_pw, D), f32),
        pltpu.VMEM((neg_pw,), i32),
        pltpu.VMEM((neg_pw,), i32),
        pltpu.VMEM((neg_pw,), i32),
        pltpu.VMEM((NBUF, C, D), f32),
        pltpu.SemaphoreType.DMA,
        pltpu.SemaphoreType.DMA((NBUF,)),
        pltpu.SemaphoreType.DMA((NBUF,)),
    ]

    def body(node_emb, short_emb, edge_flat, u_idx, v_idx, et,
             u_dec, v_dec, posu_n, posu_e, posv_n, posv_e,
             u_rand, v_rand, ret, nodes,
             u_reps_o, v_reps_o, u_edge_o, v_edge_o,
             u_pos_o, v_pos_o, u_neg_o, v_neg_o,
             nodes_v, ui_v, vi_v, et_v, eu_v, ev_v, du_v, dv_v,
             nru, sru, eru, nrv, srv, erv, rep_b, repe_b,
             sa_v, sb_v, sf_v, ring, asem, gsem, wsem):
        wid = lax.axis_index("s") * NC + lax.axis_index("c")
        ebase = pl.multiple_of(wid * e_pw, 8)

        pltpu.sync_copy(nodes, nodes_v)
        pltpu.sync_copy(u_idx.at[pl.ds(ebase, e_pw)], ui_v)
        pltpu.sync_copy(v_idx.at[pl.ds(ebase, e_pw)], vi_v)
        pltpu.sync_copy(et.at[pl.ds(ebase, e_pw)], et_v)
        pltpu.sync_copy(u_dec.at[pl.ds(ebase, e_pw)], du_v)
        pltpu.sync_copy(v_dec.at[pl.ds(ebase, e_pw)], dv_v)
        for i in range(e_pw // L):
            sl = pl.ds(i * L, L)
            e16 = et_v[sl]
            eu_v[sl] = ui_v[sl] * 4 + e16
            ev_v[sl] = vi_v[sl] * 4 + e16
        a_copies = [
            pltpu.async_copy(node_emb.at[ui_v], nru, asem),
            pltpu.async_copy(short_emb.at[ui_v], sru, asem),
            pltpu.async_copy(edge_flat.at[eu_v], eru, asem),
            pltpu.async_copy(node_emb.at[vi_v], nrv, asem),
            pltpu.async_copy(short_emb.at[vi_v], srv, asem),
            pltpu.async_copy(edge_flat.at[ev_v], erv, asem),
        ]

        def gather_seg(n_hbm, e_hbm, out_hbm, rows_pw, nchunks, via_nodes,
                       drain_prev):
            gbase0 = pl.multiple_of(wid * rows_pw, 8)
            pltpu.sync_copy(n_hbm.at[pl.ds(gbase0, rows_pw)],
                            sa_v.at[pl.ds(0, rows_pw)])
            pltpu.sync_copy(e_hbm.at[pl.ds(gbase0, rows_pw)],
                            sb_v.at[pl.ds(0, rows_pw)])

            def fidx_body(i, carry):
                sl = pl.ds(i * L, L)
                n16 = sa_v[sl]
                if via_nodes:
                    n16 = plsc.load_gather(nodes_v, [n16])
                sf_v[sl] = n16 * 4 + sb_v[sl]
                return carry
            lax.fori_loop(0, rows_pw // L, fidx_body, 0)

            def fire(t, b):
                pltpu.async_copy(
                    edge_flat.at[sf_v.at[pl.ds(t * C, C)]],
                    ring.at[b], gsem.at[b])

            def wait_gather(b):
                pltpu.make_async_copy(
                    edge_flat.at[sf_v.at[pl.ds(0, C)]],
                    ring.at[b], gsem.at[b]).wait()

            def fire_writeout(t, b):
                pltpu.async_copy(
                    ring.at[b], out_hbm.at[pl.ds(gbase0 + t * C, C)],
                    wsem.at[b])

            def drain_writeout(dst_hbm, b):
                pltpu.make_async_copy(
                    ring.at[b], dst_hbm.at[pl.ds(0, C)], wsem.at[b]).wait()

            for b in range(NBUF):
                if drain_prev is not None:
                    drain_writeout(drain_prev, b)
                fire(b, b)

            def ring_body(g, carry):
                for b in range(NBUF):
                    t = g * NBUF + b
                    wait_gather(b)
                    fire_writeout(t, b)

                    @pl.when(t + NBUF < nchunks)
                    def _():
                        drain_writeout(out_hbm, b)
                        fire(t + NBUF, b)
                return carry
            lax.fori_loop(0, nchunks // NBUF, ring_body, 0)

        gather_seg(posu_n, posu_e, u_pos_o, pos_pw, n_pos_chunks, False, None)
        gather_seg(posv_n, posv_e, v_pos_o, pos_pw, n_pos_chunks, False,
                   u_pos_o)
        gather_seg(u_rand, ret, u_neg_o, neg_pw, n_neg_chunks, True, v_pos_o)
        gather_seg(v_rand, ret, v_neg_o, neg_pw, n_neg_chunks, True, u_neg_o)
        for b in range(NBUF):
            pltpu.make_async_copy(ring.at[b], v_neg_o.at[pl.ds(0, C)],
                                  wsem.at[b]).wait()

        for c in a_copies:
            c.wait()

        def reps_compute(dec_v, nrows, srows, erows, reps_hbm, edge_hbm):
            def row_body(r, carry):
                db = plsc.load_gather(dec_v, [jnp.full((L,), r, i32)])
                for c in range(D // L):
                    sl = pl.ds(c * L, L)
                    ur = nrows[r, sl] + srows[r, sl] * db
                    rep_b[r, sl] = ur
                    repe_b[r, sl] = (ur + erows[r, sl]) * 0.5
                return carry
            lax.fori_loop(0, e_pw, row_body, 0)
            pltpu.sync_copy(rep_b, reps_hbm.at[pl.ds(ebase, e_pw)])
            pltpu.sync_copy(repe_b, edge_hbm.at[pl.ds(ebase, e_pw)])

        reps_compute(du_v, nru, sru, eru, u_reps_o, u_edge_o)
        reps_compute(dv_v, nrv, srv, erv, v_reps_o, v_edge_o)

    return pl.kernel(body, out_type=out_type, mesh=mesh,
                     scratch_types=scratch_types,
                     compiler_params=pltpu.CompilerParams(
                         needs_layout_passes=False))


def kernel(edges, walks, walks_edge_types, nodes, batch_size, n_positive,
           repeat_edge_types, u_time_delta, v_time_delta, u_pos_reps_mask,
           v_pos_reps_mask, u_pos_reps_loss_mask, v_pos_reps_loss_mask,
           node_emb_w, short_emb_w, edge_embedding, alpha, node_types_arr):
    B = walks.shape[0]
    P = walks.shape[2] * walks.shape[3]
    NNODES, NET, D = edge_embedding.shape
    NEG = N_NEG * P

    u_idx = edges[:, 0]
    v_idx = edges[:, 1]
    et = edges[:, 2]

    u_sig = jax.nn.sigmoid(alpha[node_types_arr[u_idx]])
    v_sig = jax.nn.sigmoid(alpha[node_types_arr[v_idx]])
    u_dec = 1.0 / jnp.log(2.7183 + u_sig * u_time_delta)
    v_dec = 1.0 / jnp.log(2.7183 + v_sig * v_time_delta)

    posu_n = walks[:, 0].reshape(-1)
    posv_n = walks[:, 1].reshape(-1)
    posu_e = walks_edge_types[:, 0].reshape(-1)
    posv_e = walks_edge_types[:, 1].reshape(-1)

    kneg = jax.random.key(123)
    k1, k2 = jax.random.split(kneg)
    u_rand = jax.random.randint(k1, (B * NEG,), 0, nodes.shape[0])
    v_rand = jax.random.randint(k2, (B * NEG,), 0, nodes.shape[0])
    ret = repeat_edge_types.reshape(-1)

    edge_flat = edge_embedding.reshape(NNODES * NET, D)

    sck = _build_sc_kernel(B, P, NNODES, D, NEG)
    (u_reps, v_reps, u_edge, v_edge, u_pos, v_pos, u_neg, v_neg) = sck(
        node_emb_w, short_emb_w, edge_flat,
        u_idx.astype(jnp.int32), v_idx.astype(jnp.int32),
        et.astype(jnp.int32), u_dec, v_dec,
        posu_n.astype(jnp.int32), posu_e.astype(jnp.int32),
        posv_n.astype(jnp.int32), posv_e.astype(jnp.int32),
        u_rand.astype(jnp.int32), v_rand.astype(jnp.int32),
        ret.astype(jnp.int32), nodes.astype(jnp.int32))

    return (u_reps, v_reps,
            u_pos.reshape(B, P, D), v_pos.reshape(B, P, D),
            u_neg.reshape(B, NEG, D), v_neg.reshape(B, NEG, D),
            n_positive, u_pos_reps_mask, v_pos_reps_mask,
            u_reps, v_reps, u_pos_reps_loss_mask, v_pos_reps_loss_mask,
            u_edge, v_edge)
